# baseline (speedup 1.0000x reference)
import functools

import jax
import jax.numpy as jnp
from jax import lax
from jax.experimental import pallas as pl
from jax.experimental.pallas import tpu as pltpu
from jax.experimental.pallas import tpu_sc as plsc

N_ROWS = 1_000_000
K = 64
B = 16384

_CHUNK = 128
_L = 16


def _build():
    info = plsc.get_sparse_core_info()
    nw = info.num_cores * info.num_subcores
    b_per_w = B // nw
    n_chunks = b_per_w // _CHUNK
    n_groups = b_per_w // _L
    mesh = plsc.VectorSubcoreMesh(core_axis_name="c", subcore_axis_name="s")

    @functools.partial(
        pl.kernel,
        mesh=mesh,
        out_type=(
            jax.ShapeDtypeStruct((B * K,), jnp.float32),
            jax.ShapeDtypeStruct((B * K,), jnp.float32),
        ),
        scratch_types=[
            pltpu.VMEM((b_per_w,), jnp.int32),
            pltpu.VMEM((b_per_w,), jnp.int32),
            pltpu.VMEM((b_per_w, 2 * K), jnp.float32),
            pltpu.VMEM((b_per_w * K,), jnp.float32),
            pltpu.SemaphoreType.DMA,
        ],
        compiler_params=pltpu.CompilerParams(needs_layout_passes=False),
    )
    def k(idx_hbm, mu_hbm, ls_hbm, mu_out, ls_out, idx_v, g_v, buf, out_v, sem):
        wid = lax.axis_index("s") * info.num_cores + lax.axis_index("c")
        base = wid * b_per_w
        pltpu.sync_copy(idx_hbm.at[pl.ds(base, b_per_w)], idx_v)
        for i in range(n_groups):
            v = idx_v[pl.ds(i * _L, _L)]
            g_v[pl.ds(i * _L, _L)] = v >> 1

        iota = lax.iota(jnp.int32, _L)

        for table, out_hbm in ((mu_hbm, mu_out), (ls_hbm, ls_out)):
            copies = []
            for j in range(n_chunks):
                o = j * _CHUNK
                copies.append(
                    pltpu.async_copy(
                        table.at[g_v.at[pl.ds(o, _CHUNK)]],
                        buf.at[pl.ds(o, _CHUNK)],
                        sem,
                    )
                )
            for c in copies:
                c.wait()

            def body(g, _):
                v = idx_v[pl.ds(g * _L, _L)]
                col0 = (v & 1) << 6
                row = iota + g * _L
                dst0 = (iota + g * _L) * K
                for j in range(K):
                    x = plsc.load_gather(buf, [row, col0 + j])
                    plsc.store_scatter(out_v, [dst0 + j], x)
                return _

            lax.fori_loop(0, n_groups, body, None)
            pltpu.sync_copy(out_v, out_hbm.at[pl.ds(base * K, b_per_w * K)])

    return k


_gather = _build()


def kernel(indices, mu, log_sigma):
    mu2 = mu.reshape(N_ROWS // 2, 2 * K)
    ls2 = log_sigma.reshape(N_ROWS // 2, 2 * K)
    mu_out, ls_out = _gather(indices.astype(jnp.int32), mu2, ls2)
    return (mu_out.reshape(B, K), ls_out.reshape(B, K))


# speedup vs baseline: 1.6874x; 1.6874x over previous
import functools

import jax
import jax.numpy as jnp
from jax import lax
from jax.experimental import pallas as pl
from jax.experimental.pallas import tpu as pltpu
from jax.experimental.pallas import tpu_sc as plsc

N_ROWS = 1_000_000
K = 64
B = 16384

_L = 16
_WAVE = 256


def _build():
    info = plsc.get_sparse_core_info()
    nw = info.num_cores * info.num_subcores
    b_per_w = B // nw
    n_waves = b_per_w // _WAVE
    n_groups = _WAVE // _L
    mesh = plsc.VectorSubcoreMesh(core_axis_name="c", subcore_axis_name="s")

    @functools.partial(
        pl.kernel,
        mesh=mesh,
        out_type=(
            jax.ShapeDtypeStruct((B, K), jnp.float32),
            jax.ShapeDtypeStruct((B, K), jnp.float32),
        ),
        scratch_types=[
            pltpu.VMEM((b_per_w,), jnp.int32),
            pltpu.VMEM((_WAVE, K), jnp.float32),
            pltpu.VMEM((_WAVE, K), jnp.float32),
            pltpu.SemaphoreType.DMA,
            pltpu.SemaphoreType.DMA,
        ],
        compiler_params=pltpu.CompilerParams(needs_layout_passes=False),
    )
    def k(idx_hbm, mu_hbm, ls_hbm, mu_out, ls_out, idx_v, mu_v, ls_v, sem, sem2):
        wid = lax.axis_index("s") * info.num_cores + lax.axis_index("c")
        base = wid * b_per_w
        pltpu.sync_copy(idx_hbm.at[pl.ds(base, b_per_w)], idx_v)
        lane = lax.iota(jnp.int32, _L)

        def wave(w, _):
            o = w * _WAVE

            def fire(g, _):
                v = idx_v[pl.ds(o + g * _L, _L)]
                for j in range(_L):
                    r = jnp.max(jnp.where(lane == j, v, 0))
                    slot = g * _L + j
                    pltpu.async_copy(mu_hbm.at[r], mu_v.at[slot], sem)
                    pltpu.async_copy(ls_hbm.at[r], ls_v.at[slot], sem2)
                return _

            lax.fori_loop(0, n_groups, fire, None)

            def drain(i, _):
                pltpu.make_async_copy(mu_hbm.at[0], mu_v.at[0], sem).wait()
                pltpu.make_async_copy(ls_hbm.at[0], ls_v.at[0], sem2).wait()
                return _

            lax.fori_loop(0, _WAVE, drain, None)
            pltpu.sync_copy(mu_v, mu_out.at[pl.ds(base + o, _WAVE)])
            pltpu.sync_copy(ls_v, ls_out.at[pl.ds(base + o, _WAVE)])
            return _

        lax.fori_loop(0, n_waves, wave, None)

    return k


_gather = _build()


def kernel(indices, mu, log_sigma):
    return _gather(indices.astype(jnp.int32), mu, log_sigma)


# speedup vs baseline: 1.6914x; 1.0023x over previous
import functools

import jax
import jax.numpy as jnp
from jax import lax
from jax.experimental import pallas as pl
from jax.experimental.pallas import tpu as pltpu
from jax.experimental.pallas import tpu_sc as plsc

N_ROWS = 1_000_000
K = 64
B = 16384

_WAVE = 256
_UNROLL = 8


def _build():
    info = plsc.get_sparse_core_info()
    nc, ns = info.num_cores, info.num_subcores
    nw = nc * ns
    b_per_w = B // nw
    n_waves = b_per_w // _WAVE
    mesh = plsc.VectorSubcoreMesh(core_axis_name="c", subcore_axis_name="s")

    @functools.partial(
        pl.kernel,
        mesh=mesh,
        out_type=(
            jax.ShapeDtypeStruct((B, K), jnp.float32),
            jax.ShapeDtypeStruct((B, K), jnp.float32),
        ),
        scratch_types=[
            pltpu.VMEM_SHARED((ns, b_per_w), jnp.int32),
            pltpu.SMEM((b_per_w,), jnp.int32),
            pltpu.VMEM((_WAVE, K), jnp.float32),
            pltpu.VMEM((_WAVE, K), jnp.float32),
            pltpu.SemaphoreType.DMA,
            pltpu.SemaphoreType.DMA,
        ],
        compiler_params=pltpu.CompilerParams(needs_layout_passes=False),
    )
    def k(idx_hbm, mu_hbm, ls_hbm, mu_out, ls_out, idx_sh, idx_s, mu_v, ls_v, sem, sem2):
        cid = lax.axis_index("c")
        sid = lax.axis_index("s")
        wid = sid * nc + cid
        base = wid * b_per_w
        pltpu.sync_copy(idx_hbm.at[pl.ds(base, b_per_w)], idx_sh.at[sid])
        pltpu.sync_copy(idx_sh.at[sid], idx_s)

        def wave(w, _):
            o = w * _WAVE

            def fire(g, _):
                for j in range(_UNROLL):
                    i = g * _UNROLL + j
                    r = idx_s[o + i]
                    pltpu.async_copy(mu_hbm.at[r], mu_v.at[i], sem)
                    pltpu.async_copy(ls_hbm.at[r], ls_v.at[i], sem2)
                return _

            lax.fori_loop(0, _WAVE // _UNROLL, fire, None)

            def drain(i, _):
                pltpu.make_async_copy(mu_hbm.at[0], mu_v.at[0], sem).wait()
                pltpu.make_async_copy(ls_hbm.at[0], ls_v.at[0], sem2).wait()
                return _

            lax.fori_loop(0, _WAVE, drain, None)
            pltpu.sync_copy(mu_v, mu_out.at[pl.ds(base + o, _WAVE)])
            pltpu.sync_copy(ls_v, ls_out.at[pl.ds(base + o, _WAVE)])
            return _

        lax.fori_loop(0, n_waves, wave, None)

    return k


_gather = _build()


def kernel(indices, mu, log_sigma):
    return _gather(indices.astype(jnp.int32), mu, log_sigma)


# speedup vs baseline: 1.6916x; 1.0001x over previous
import functools

import jax
import jax.numpy as jnp
from jax import lax
from jax.experimental import pallas as pl
from jax.experimental.pallas import tpu as pltpu
from jax.experimental.pallas import tpu_sc as plsc

N_ROWS = 1_000_000
K = 64
B = 16384

_WAVE = 256
_UNROLL = 8


def _build():
    info = plsc.get_sparse_core_info()
    nc, ns = info.num_cores, info.num_subcores
    nw = nc * ns
    b_per_w = B // nw
    n_waves = b_per_w // _WAVE
    mesh = plsc.VectorSubcoreMesh(core_axis_name="c", subcore_axis_name="s")

    @functools.partial(
        pl.kernel,
        mesh=mesh,
        out_type=(
            jax.ShapeDtypeStruct((B, K), jnp.float32),
            jax.ShapeDtypeStruct((B, K), jnp.float32),
        ),
        scratch_types=[
            pltpu.VMEM_SHARED((ns, b_per_w), jnp.int32),
            pltpu.SMEM((b_per_w,), jnp.int32),
            pltpu.VMEM((_WAVE, K), jnp.float32),
            pltpu.VMEM((_WAVE, K), jnp.float32),
            pltpu.SemaphoreType.DMA,
            pltpu.SemaphoreType.DMA,
            pltpu.SemaphoreType.DMA,
            pltpu.SemaphoreType.DMA,
            pltpu.SemaphoreType.DMA,
            pltpu.SemaphoreType.DMA,
            pltpu.SemaphoreType.DMA,
            pltpu.SemaphoreType.DMA,
        ],
        compiler_params=pltpu.CompilerParams(needs_layout_passes=False),
    )
    def k(idx_hbm, mu_hbm, ls_hbm, mu_out, ls_out, idx_sh, idx_s, mu_v, ls_v,
          sa0, sa1, sa2, sa3, sb0, sb1, sb2, sb3):
        sems_a = (sa0, sa1, sa2, sa3)
        sems_b = (sb0, sb1, sb2, sb3)
        cid = lax.axis_index("c")
        sid = lax.axis_index("s")
        wid = sid * nc + cid
        base = wid * b_per_w
        pltpu.sync_copy(idx_hbm.at[pl.ds(base, b_per_w)], idx_sh.at[sid])
        pltpu.sync_copy(idx_sh.at[sid], idx_s)

        def wave(w, _):
            o = w * _WAVE

            def fire(g, _):
                for j in range(_UNROLL):
                    i = g * _UNROLL + j
                    r = idx_s[o + i]
                    pltpu.async_copy(mu_hbm.at[r], mu_v.at[i], sems_a[j % 4])
                    pltpu.async_copy(ls_hbm.at[r], ls_v.at[i], sems_b[j % 4])
                return _

            lax.fori_loop(0, _WAVE // _UNROLL, fire, None)

            def drain(i, _):
                for j in range(4):
                    pltpu.make_async_copy(mu_hbm.at[0], mu_v.at[0], sems_a[j]).wait()
                    pltpu.make_async_copy(ls_hbm.at[0], ls_v.at[0], sems_b[j]).wait()
                return _

            lax.fori_loop(0, _WAVE // 4, drain, None)
            pltpu.sync_copy(mu_v, mu_out.at[pl.ds(base + o, _WAVE)])
            pltpu.sync_copy(ls_v, ls_out.at[pl.ds(base + o, _WAVE)])
            return _

        lax.fori_loop(0, n_waves, wave, None)

    return k


_gather = _build()


def kernel(indices, mu, log_sigma):
    return _gather(indices.astype(jnp.int32), mu, log_sigma)
